# baseline (speedup 1.0000x reference)
import functools

import jax
import jax.numpy as jnp
from jax import lax
from jax.experimental import pallas as pl
from jax.experimental.pallas import tpu as pltpu
from jax.experimental.pallas import tpu_sc as plsc

N_NODES = 10000
D = 128
NP = 10240
NC = 2
NS = 16
NW = NC * NS
CH = 128
CPW = 80
EPAD = NW * CPW * CH
RPT = NP // NS

_mesh = plsc.VectorSubcoreMesh(
    core_axis_name="c", subcore_axis_name="s", num_cores=NC, num_subcores=NS
)


@functools.partial(
    pl.kernel,
    out_type=jax.ShapeDtypeStruct((NC, NP), jnp.float32),
    mesh=_mesh,
    scratch_types=[
        pltpu.VMEM((CPW, CH), jnp.int32),
        pltpu.VMEM((CH,), jnp.float32),
        pltpu.VMEM((RPT,), jnp.float32),
        pltpu.VMEM_SHARED((NP,), jnp.float32),
    ],
)
def _deg_kernel(e_hbm, degp_hbm, idx_v, ones_v, z_v, deg_sh):
    cid = lax.axis_index("c")
    sid = lax.axis_index("s")
    wid = sid * NC + cid

    def fill_zero(i, carry):
        z_v[pl.ds(i * 16, 16)] = jnp.zeros((16,), jnp.float32)
        return carry

    lax.fori_loop(0, RPT // 16, fill_zero, 0)
    for j in range(CH // 16):
        ones_v[pl.ds(j * 16, 16)] = jnp.ones((16,), jnp.float32)
    pltpu.sync_copy(z_v, deg_sh.at[pl.ds(sid * RPT, RPT)])
    pltpu.sync_copy(e_hbm.at[1, wid], idx_v)
    plsc.subcore_barrier()

    def step(c, carry):
        pltpu.sync_copy(ones_v, deg_sh.at[idx_v.at[c]], add=True)
        return carry

    lax.fori_loop(0, CPW, step, 0)
    plsc.subcore_barrier()
    pltpu.sync_copy(
        deg_sh.at[pl.ds(sid * RPT, RPT)],
        degp_hbm.at[cid, pl.ds(sid * RPT, RPT)],
    )


@functools.partial(
    pl.kernel,
    out_type=jax.ShapeDtypeStruct((NC, NP, D), jnp.float32),
    mesh=_mesh,
    scratch_types=[
        pltpu.VMEM((CPW, CH), jnp.int32),
        pltpu.VMEM((2, CH), jnp.int32),
        pltpu.VMEM((CH, D), jnp.float32),
        pltpu.VMEM((CH, D), jnp.float32),
        pltpu.VMEM_SHARED((NP, D), jnp.float32),
        pltpu.SemaphoreType.DMA,
        pltpu.SemaphoreType.DMA,
        pltpu.SemaphoreType.DMA,
        pltpu.SemaphoreType.DMA,
    ],
)
def _spmm_kernel(y_hbm, e_hbm, acc_hbm, isrc, idst, buf0, buf1,
                 acc_sh, sem0, sem1, semd0, semd1):
    cid = lax.axis_index("c")
    sid = lax.axis_index("s")
    wid = sid * NC + cid

    zeros16 = jnp.zeros((16,), jnp.float32)

    def fill_zero(i, carry):
        for j in range(D // 16):
            buf0[i, pl.ds(j * 16, 16)] = zeros16
        return carry

    lax.fori_loop(0, CH, fill_zero, 0)

    def zero_acc(k, carry):
        pltpu.sync_copy(buf0, acc_sh.at[pl.ds(sid * RPT + k * CH, CH)])
        return carry

    lax.fori_loop(0, RPT // CH, zero_acc, 0)
    pltpu.sync_copy(e_hbm.at[1, wid], isrc)
    plsc.subcore_barrier()

    pltpu.async_copy(e_hbm.at[0, wid, 0], idst.at[0], semd0)
    pltpu.async_copy(e_hbm.at[0, wid, 1], idst.at[1], semd1)
    pltpu.async_copy(y_hbm.at[isrc.at[0]], buf0, sem0)
    pltpu.async_copy(y_hbm.at[isrc.at[1]], buf1, sem1)

    def pair(g, carry):
        c0 = 2 * g
        pltpu.make_async_copy(y_hbm.at[isrc.at[c0]], buf0, sem0).wait()
        pltpu.make_async_copy(
            e_hbm.at[0, wid, c0], idst.at[0], semd0
        ).wait()
        pltpu.sync_copy(buf0, acc_sh.at[idst.at[0]], add=True)

        @pl.when(g < CPW // 2 - 1)
        def _():
            pltpu.async_copy(e_hbm.at[0, wid, c0 + 2], idst.at[0], semd0)
            pltpu.async_copy(y_hbm.at[isrc.at[c0 + 2]], buf0, sem0)

        pltpu.make_async_copy(y_hbm.at[isrc.at[c0 + 1]], buf1, sem1).wait()
        pltpu.make_async_copy(
            e_hbm.at[0, wid, c0 + 1], idst.at[1], semd1
        ).wait()
        pltpu.sync_copy(buf1, acc_sh.at[idst.at[1]], add=True)

        @pl.when(g < CPW // 2 - 1)
        def _():
            pltpu.async_copy(e_hbm.at[0, wid, c0 + 3], idst.at[1], semd1)
            pltpu.async_copy(y_hbm.at[isrc.at[c0 + 3]], buf1, sem1)

        return carry

    lax.fori_loop(0, CPW // 2, pair, 0)
    plsc.subcore_barrier()
    pltpu.sync_copy(
        acc_sh.at[pl.ds(sid * RPT, RPT)],
        acc_hbm.at[cid, pl.ds(sid * RPT, RPT)],
    )


def _t1_body(da_ref, db_ref, x_ref, dis_ref, dis2_ref, y_ref):
    deg = da_ref[...] + db_ref[...] + 1.0
    dis = lax.rsqrt(deg)
    dis_ref[...] = dis
    dis2_ref[...] = dis * dis
    y_ref[: N_NODES, :] = dis[: N_NODES, :] * x_ref[...]
    y_ref[N_NODES :, :] = jnp.zeros((NP - N_NODES, D), jnp.float32)


_t1 = pl.pallas_call(
    _t1_body,
    out_shape=(
        jax.ShapeDtypeStruct((NP, 1), jnp.float32),
        jax.ShapeDtypeStruct((NP, 1), jnp.float32),
        jax.ShapeDtypeStruct((NP, D), jnp.float32),
    ),
)


def _combine_body(s_ref, acc_ref, y_ref, o_ref):
    o_ref[...] = s_ref[...] * (acc_ref[0] + acc_ref[1] + y_ref[...])


_combine = pl.pallas_call(
    _combine_body,
    out_shape=jax.ShapeDtypeStruct((NP, D), jnp.float32),
)


def _combine_final_body(s_ref, acc_ref, y_ref, o_ref):
    n = N_NODES
    o_ref[...] = s_ref[:n, :] * (
        acc_ref[0, :n, :] + acc_ref[1, :n, :] + y_ref[:n, :]
    )


_combine_final = pl.pallas_call(
    _combine_final_body,
    out_shape=jax.ShapeDtypeStruct((N_NODES, D), jnp.float32),
)


def kernel(edge_index, x):
    e = edge_index.shape[1]
    e4 = jnp.pad(
        edge_index.astype(jnp.int32),
        ((0, 0), (0, EPAD - e)),
        constant_values=N_NODES,
    ).reshape(2, NW, CPW, CH)
    degp = _deg_kernel(e4)
    da = degp[0].reshape(NP, 1)
    db = degp[1].reshape(NP, 1)
    dis, dis2, y1 = _t1(da, db, x)

    acc1 = _spmm_kernel(y1, e4)
    y2 = _combine(dis2, acc1, y1)
    acc2 = _spmm_kernel(y2, e4)
    return _combine_final(dis, acc2, y2)


# speedup vs baseline: 3.4165x; 3.4165x over previous
import functools

import numpy as np

import jax
import jax.numpy as jnp
from jax import lax
from jax.experimental import pallas as pl
from jax.experimental.pallas import tpu as pltpu
from jax.experimental.pallas import tpu_sc as plsc

N_NODES = 10000
D = 128
NP = 10240
NC = 2
NS = 16
NW = NC * NS
CH = 128
CPW = 80
EPAD = NW * CPW * CH
RPT = NP // NS

_mesh = plsc.VectorSubcoreMesh(
    core_axis_name="c", subcore_axis_name="s", num_cores=NC, num_subcores=NS
)


@functools.partial(
    pl.kernel,
    out_type=jax.ShapeDtypeStruct((NC, NP), jnp.float32),
    mesh=_mesh,
    scratch_types=[
        pltpu.VMEM((CPW, CH), jnp.int32),
        pltpu.VMEM((CH,), jnp.float32),
        pltpu.VMEM((RPT,), jnp.float32),
        pltpu.VMEM_SHARED((NP,), jnp.float32),
    ],
)
def _deg_kernel(e_hbm, degp_hbm, idx_v, ones_v, z_v, deg_sh):
    cid = lax.axis_index("c")
    sid = lax.axis_index("s")
    wid = sid * NC + cid

    def fill_zero(i, carry):
        z_v[pl.ds(i * 16, 16)] = jnp.zeros((16,), jnp.float32)
        return carry

    lax.fori_loop(0, RPT // 16, fill_zero, 0)
    for j in range(CH // 16):
        ones_v[pl.ds(j * 16, 16)] = jnp.ones((16,), jnp.float32)
    pltpu.sync_copy(z_v, deg_sh.at[pl.ds(sid * RPT, RPT)])
    pltpu.sync_copy(e_hbm.at[1, wid], idx_v)
    plsc.subcore_barrier()

    def step(c, carry):
        pltpu.sync_copy(ones_v, deg_sh.at[idx_v.at[c]], add=True)
        return carry

    lax.fori_loop(0, CPW, step, 0)
    plsc.subcore_barrier()
    pltpu.sync_copy(
        deg_sh.at[pl.ds(sid * RPT, RPT)],
        degp_hbm.at[cid, pl.ds(sid * RPT, RPT)],
    )


@functools.partial(
    pl.kernel,
    out_type=jax.ShapeDtypeStruct((NC, NP, D), jnp.float32),
    mesh=_mesh,
    scratch_types=[
        pltpu.VMEM((CPW, CH), jnp.int32),
        pltpu.VMEM((2, CH), jnp.int32),
        pltpu.VMEM((CH, D), jnp.float32),
        pltpu.VMEM((CH, D), jnp.float32),
        pltpu.VMEM_SHARED((NP, D), jnp.float32),
        pltpu.SemaphoreType.DMA,
        pltpu.SemaphoreType.DMA,
        pltpu.SemaphoreType.DMA,
        pltpu.SemaphoreType.DMA,
    ],
)
def _spmm_kernel(y_hbm, e_hbm, acc_hbm, isrc, idst, buf0, buf1,
                 acc_sh, sem0, sem1, semd0, semd1):
    cid = lax.axis_index("c")
    sid = lax.axis_index("s")
    wid = sid * NC + cid

    zeros16 = jnp.zeros((16,), jnp.float32)

    def fill_zero(i, carry):
        for j in range(D // 16):
            buf0[i, pl.ds(j * 16, 16)] = zeros16
        return carry

    lax.fori_loop(0, CH, fill_zero, 0)

    def zero_acc(k, carry):
        pltpu.sync_copy(buf0, acc_sh.at[pl.ds(sid * RPT + k * CH, CH)])
        return carry

    lax.fori_loop(0, RPT // CH, zero_acc, 0)
    pltpu.sync_copy(e_hbm.at[1, wid], isrc)
    plsc.subcore_barrier()

    pltpu.async_copy(e_hbm.at[0, wid, 0], idst.at[0], semd0)
    pltpu.async_copy(e_hbm.at[0, wid, 1], idst.at[1], semd1)
    pltpu.async_copy(y_hbm.at[isrc.at[0]], buf0, sem0)
    pltpu.async_copy(y_hbm.at[isrc.at[1]], buf1, sem1)

    def pair(g, carry):
        c0 = 2 * g
        pltpu.make_async_copy(y_hbm.at[isrc.at[c0]], buf0, sem0).wait()
        pltpu.make_async_copy(
            e_hbm.at[0, wid, c0], idst.at[0], semd0
        ).wait()
        pltpu.sync_copy(buf0, acc_sh.at[idst.at[0]], add=True)

        @pl.when(g < CPW // 2 - 1)
        def _():
            pltpu.async_copy(e_hbm.at[0, wid, c0 + 2], idst.at[0], semd0)
            pltpu.async_copy(y_hbm.at[isrc.at[c0 + 2]], buf0, sem0)

        pltpu.make_async_copy(y_hbm.at[isrc.at[c0 + 1]], buf1, sem1).wait()
        pltpu.make_async_copy(
            e_hbm.at[0, wid, c0 + 1], idst.at[1], semd1
        ).wait()
        pltpu.sync_copy(buf1, acc_sh.at[idst.at[1]], add=True)

        @pl.when(g < CPW // 2 - 1)
        def _():
            pltpu.async_copy(e_hbm.at[0, wid, c0 + 3], idst.at[1], semd1)
            pltpu.async_copy(y_hbm.at[isrc.at[c0 + 3]], buf1, sem1)

        return carry

    lax.fori_loop(0, CPW // 2, pair, 0)
    plsc.subcore_barrier()
    pltpu.sync_copy(
        acc_sh.at[pl.ds(sid * RPT, RPT)],
        acc_hbm.at[cid, pl.ds(sid * RPT, RPT)],
    )


def _t1_body(da_ref, db_ref, x_ref, dis_ref, dis2_ref, y_ref):
    deg = da_ref[...] + db_ref[...] + 1.0
    dis = lax.rsqrt(deg)
    dis_ref[...] = dis
    dis2_ref[...] = dis * dis
    y_ref[: N_NODES, :] = dis[: N_NODES, :] * x_ref[...]
    y_ref[N_NODES :, :] = jnp.zeros((NP - N_NODES, D), jnp.float32)


_t1 = pl.pallas_call(
    _t1_body,
    out_shape=(
        jax.ShapeDtypeStruct((NP, 1), jnp.float32),
        jax.ShapeDtypeStruct((NP, 1), jnp.float32),
        jax.ShapeDtypeStruct((NP, D), jnp.float32),
    ),
)


def _combine_body(s_ref, acc_ref, y_ref, o_ref):
    o_ref[...] = s_ref[...] * (acc_ref[0] + acc_ref[1] + y_ref[...])


_combine = pl.pallas_call(
    _combine_body,
    out_shape=jax.ShapeDtypeStruct((NP, D), jnp.float32),
)


def _combine_final_body(s_ref, acc_ref, y_ref, o_ref):
    n = N_NODES
    o_ref[...] = s_ref[:n, :] * (
        acc_ref[0, :n, :] + acc_ref[1, :n, :] + y_ref[:n, :]
    )


_combine_final = pl.pallas_call(
    _combine_final_body,
    out_shape=jax.ShapeDtypeStruct((N_NODES, D), jnp.float32),
)


_E_REAL = 320000
_DUMMY = np.broadcast_to(
    N_NODES + (np.arange(EPAD - _E_REAL) % (NP - N_NODES)), (2, EPAD - _E_REAL)
).astype(np.int32)


def kernel(edge_index, x):
    e4 = jnp.concatenate(
        [edge_index.astype(jnp.int32), jnp.asarray(_DUMMY)], axis=1
    ).reshape(2, NW, CPW, CH)
    degp = _deg_kernel(e4)
    da = degp[0].reshape(NP, 1)
    db = degp[1].reshape(NP, 1)
    dis, dis2, y1 = _t1(da, db, x)

    acc1 = _spmm_kernel(y1, e4)
    y2 = _combine(dis2, acc1, y1)
    acc2 = _spmm_kernel(y2, e4)
    return _combine_final(dis, acc2, y2)
